# baseline (speedup 1.0000x reference)
import jax
import jax.numpy as jnp
from jax.experimental import pallas as pl
from jax.experimental.pallas import tpu as pltpu

BS, L, DIM = 4, 512, 128
IG = 128
NG = L // IG
CH = 8
NCH = IG // CH


def _gcn_body(q_ref, qg_ref, qT_ref, qgT_ref, depT_ref, a_ref, b_ref,
              w2bd_ref, wdT_ref, outT_ref, prei_ref, prej_ref, h8_ref,
              gacc_ref, hjT_ref):
    @pl.when(pl.program_id(1) == 0)
    def _():
        prej_ref[...] = jnp.dot(q_ref[0], b_ref[...],
                                preferred_element_type=jnp.float32)
        hjT_ref[...] = jnp.dot(wdT_ref[...], qT_ref[0],
                               preferred_element_type=jnp.float32)

    prei_ref[...] = jnp.dot(qg_ref[0], a_ref[...],
                            preferred_element_type=jnp.float32)
    gacc_ref[...] = jnp.zeros((L, IG), jnp.float32)

    u_iota = jax.lax.broadcasted_iota(jnp.int32, (CH, IG), 0)
    l_iota = jax.lax.broadcasted_iota(jnp.int32, (CH, IG), 1)

    def step(k, _):
        kb = jax.lax.rem(k, NCH)
        buf = jax.lax.rem(k, 2)
        prej = prej_ref[...]
        for u in range(CH):
            r = prei_ref[pl.ds(kb * CH + u, 1), :]
            h8_ref[buf, :, DIM * u:DIM * (u + 1)] = (
                jnp.maximum(prej + r, 0.0).astype(jnp.bfloat16))
        kr = k - 1
        rbuf = jax.lax.rem(jnp.maximum(kr, 0), 2)
        tmp = jnp.dot(h8_ref[rbuf], w2bd_ref[...],
                      preferred_element_type=jnp.float32)
        hit = jnp.logical_and(l_iota == CH * kr + u_iota, kr >= 0)
        place = hit.astype(jnp.float32)
        gacc_ref[...] += jnp.dot(tmp, place,
                                 preferred_element_type=jnp.float32)
        return 0

    jax.lax.fori_loop(0, NCH + 1, step, 0)

    maskT = depT_ref[0] > 0
    T = jnp.where(maskT, gacc_ref[...], jnp.float32(-100.0))
    m = jnp.max(T, axis=0, keepdims=True)
    e = jnp.exp(T - m)
    r = 1.0 / jnp.sum(e, axis=0, keepdims=True)
    betam = e * r * maskT.astype(jnp.float32)

    aggT = jnp.dot(hjT_ref[...], betam,
                   preferred_element_type=jnp.float32)
    outT_ref[0] = jnp.maximum(qgT_ref[0] + aggT, 0.0)


def kernel(queries, wordlens, syntactic_dep, W1, W2, Wd):
    q = queries.astype(jnp.float32)
    qT = jnp.swapaxes(q, 1, 2)
    depT = jnp.swapaxes(syntactic_dep.astype(jnp.int32), 1, 2)
    A = W1[:DIM, :]
    B = W1[DIM:, :]
    W2bd = jnp.kron(jnp.eye(CH, dtype=jnp.float32),
                    W2).astype(jnp.bfloat16)
    WdT = jnp.swapaxes(Wd, 0, 1)

    outT = pl.pallas_call(
        _gcn_body,
        grid=(BS, NG),
        in_specs=[
            pl.BlockSpec((1, L, DIM), lambda b, g: (b, 0, 0)),
            pl.BlockSpec((1, IG, DIM), lambda b, g: (b, g, 0)),
            pl.BlockSpec((1, DIM, L), lambda b, g: (b, 0, 0)),
            pl.BlockSpec((1, DIM, IG), lambda b, g: (b, 0, g)),
            pl.BlockSpec((1, L, IG), lambda b, g: (b, 0, g)),
            pl.BlockSpec((DIM, DIM), lambda b, g: (0, 0)),
            pl.BlockSpec((DIM, DIM), lambda b, g: (0, 0)),
            pl.BlockSpec((CH * DIM, CH), lambda b, g: (0, 0)),
            pl.BlockSpec((DIM, DIM), lambda b, g: (0, 0)),
        ],
        out_specs=pl.BlockSpec((1, DIM, IG), lambda b, g: (b, 0, g)),
        out_shape=jax.ShapeDtypeStruct((BS, DIM, L), jnp.float32),
        scratch_shapes=[
            pltpu.VMEM((IG, DIM), jnp.float32),
            pltpu.VMEM((L, DIM), jnp.float32),
            pltpu.VMEM((2, L, CH * DIM), jnp.bfloat16),
            pltpu.VMEM((L, IG), jnp.float32),
            pltpu.VMEM((DIM, L), jnp.float32),
        ],
        compiler_params=pltpu.CompilerParams(
            dimension_semantics=("arbitrary", "arbitrary"),
        ),
    )(q, q, qT, qT, depT, A, B, W2bd, WdT)

    out = jnp.swapaxes(outT, 1, 2)
    return (out, wordlens, syntactic_dep)
